# baseline (speedup 1.0000x reference)
import functools

import jax
import jax.numpy as jnp
from jax import lax
from jax.experimental import pallas as pl
from jax.experimental.pallas import tpu as pltpu
from jax.experimental.pallas import tpu_sc as plsc

N = 10000
E = 320000
D = 128
NC = 2
NS = 16
B = 32
Q = 8
EPT = E // NS
EPT_PAD = 20480
NBLK = EPT_PAD // 4096
KB = 4096 // B
K = EPT_PAD // B

_mesh = plsc.VectorSubcoreMesh(core_axis_name="c", subcore_axis_name="s")


@functools.partial(
    pl.kernel,
    out_type=jax.ShapeDtypeStruct((NC, N, D), jnp.float32),
    mesh=_mesh,
    scratch_types=[
        pltpu.VMEM((2, 32, 128), jnp.int32),
        pltpu.VMEM((2, 32, 128), jnp.int32),
        pltpu.VMEM((Q, B, D), jnp.float32),
        pltpu.VMEM_SHARED((N, D), jnp.float32),
        pltpu.SemaphoreType.DMA((Q,)),
        pltpu.SemaphoreType.DMA((2,)),
        pltpu.SemaphoreType.DMA((Q,)),
    ],
)
def _agg_kernel(x_hbm, src_hbm, dst_hbm, out_hbm, src_v, dst_v, rows_v,
                acc_sh, gsem, isem, ssem):
    c = lax.axis_index("c")
    s = lax.axis_index("s")

    pltpu.sync_copy(
        x_hbm.at[pl.ds(s * 624, 624)], acc_sh.at[pl.ds(s * 624, 624)])

    @pl.when(s == 0)
    def _():
        pltpu.sync_copy(
            x_hbm.at[pl.ds(16 * 624, N - 16 * 624)],
            acc_sh.at[pl.ds(16 * 624, N - 16 * 624)],
        )

    plsc.subcore_barrier()

    pltpu.sync_copy(src_hbm.at[c, s, 0], src_v.at[0])
    pltpu.sync_copy(dst_hbm.at[c, s, 0], dst_v.at[0])
    for t in range(Q - 1):
        pltpu.async_copy(
            x_hbm.at[src_v.at[0, t // 4, pl.ds((t % 4) * B, B)]],
            rows_v.at[t], gsem.at[t])

    def body(j, carry):
        p = j % Q
        b = j // KB
        jj = j % KB
        bp = b % 2

        g = j + Q - 1
        gp = g % Q
        gb = g // KB
        gjj = g % KB
        gbp = gb % 2

        @pl.when(jnp.logical_and(jj == 0, b + 1 < NBLK))
        def _():
            pltpu.async_copy(src_hbm.at[c, s, b + 1], src_v.at[1 - bp],
                             isem.at[1 - bp])
            pltpu.async_copy(dst_hbm.at[c, s, b + 1], dst_v.at[1 - bp],
                             isem.at[1 - bp])

        @pl.when(g < K)
        def _():
            @pl.when(gjj == 0)
            def _():
                pltpu.make_async_copy(src_hbm.at[c, s, gb], src_v.at[gbp],
                                      isem.at[gbp]).wait()
                pltpu.make_async_copy(dst_hbm.at[c, s, gb], dst_v.at[gbp],
                                      isem.at[gbp]).wait()

            @pl.when(g >= Q)
            def _():
                pltpu.make_async_copy(
                    rows_v.at[gp], acc_sh.at[dst_v.at[0, 0, pl.ds(0, B)]],
                    ssem.at[gp]).wait()

            pltpu.async_copy(
                x_hbm.at[src_v.at[gbp, gjj // 4, pl.ds((gjj % 4) * B, B)]],
                rows_v.at[gp], gsem.at[gp])

        pltpu.make_async_copy(x_hbm.at[src_v.at[0, 0, pl.ds(0, B)]],
                              rows_v.at[p], gsem.at[p]).wait()
        pltpu.async_copy(
            rows_v.at[p],
            acc_sh.at[dst_v.at[bp, jj // 4, pl.ds((jj % 4) * B, B)]],
            ssem.at[p], add=True)
        return carry

    lax.fori_loop(0, K, body, 0, unroll=False)

    for t in range(Q):
        pltpu.make_async_copy(rows_v.at[t], acc_sh.at[dst_v.at[0, 0, pl.ds(0, B)]],
                              ssem.at[t]).wait()

    plsc.subcore_barrier()
    pltpu.sync_copy(
        acc_sh.at[pl.ds(s * 624, 624)],
        out_hbm.at[c, pl.ds(s * 624, 624)],
    )

    @pl.when(s == 0)
    def _():
        pltpu.sync_copy(
            acc_sh.at[pl.ds(16 * 624, N - 16 * 624)],
            out_hbm.at[c, pl.ds(16 * 624, N - 16 * 624)],
        )


BLK = 1000


def _mlp_body(h_ref, w1_ref, b1_ref, w2_ref, b2_ref, o_ref):
    w1 = w1_ref[...]
    b1 = b1_ref[...]
    rf = jnp.maximum(
        jnp.dot(h_ref[0], w1, preferred_element_type=jnp.float32) + b1, 0.0)
    rb = jnp.maximum(
        jnp.dot(h_ref[1], w1, preferred_element_type=jnp.float32) + b1, 0.0)
    o = jnp.dot(rf + rb, w2_ref[...], preferred_element_type=jnp.float32) * 0.5
    o_ref[...] = jnp.maximum(o + b2_ref[...], 0.0)


def kernel(x, edge_index, reverse_edge_index, W1, b1, W2, b2):
    ei = edge_index.astype(jnp.int32)
    rei = reverse_edge_index.astype(jnp.int32)
    npad_e = EPT_PAD - EPT
    src = jnp.stack([ei[0], rei[0]]).reshape(NC, NS, EPT)
    dst = jnp.stack([ei[1], rei[1]]).reshape(NC, NS, EPT)
    srcp = jnp.full((NC, NS, npad_e), N, dtype=jnp.int32)
    dstp = jnp.broadcast_to(jnp.arange(npad_e, dtype=jnp.int32),
                            (NC, NS, npad_e))
    src = jnp.concatenate([src, srcp], axis=2).reshape(NC, NS, NBLK, 32, 128)
    dst = jnp.concatenate([dst, dstp], axis=2).reshape(NC, NS, NBLK, 32, 128)
    x_aug = jnp.concatenate([x, jnp.zeros((8, D), jnp.float32)])

    h = _agg_kernel(x_aug, src, dst)

    out = pl.pallas_call(
        _mlp_body,
        grid=(N // BLK,),
        in_specs=[
            pl.BlockSpec((NC, BLK, D), lambda i: (0, i, 0)),
            pl.BlockSpec((D, D), lambda i: (0, 0)),
            pl.BlockSpec((1, D), lambda i: (0, 0)),
            pl.BlockSpec((D, D), lambda i: (0, 0)),
            pl.BlockSpec((1, D), lambda i: (0, 0)),
        ],
        out_specs=pl.BlockSpec((BLK, D), lambda i: (i, 0)),
        out_shape=jax.ShapeDtypeStruct((N, D), jnp.float32),
    )(h, W1, b1.reshape(1, D), W2, b2.reshape(1, D))
    return out


# speedup vs baseline: 3.0951x; 3.0951x over previous
import functools

import jax
import jax.numpy as jnp
from jax import lax
from jax.experimental import pallas as pl
from jax.experimental.pallas import tpu as pltpu
from jax.experimental.pallas import tpu_sc as plsc

N = 10000
E = 320000
D = 128
NC = 2
NS = 16
B = 32
NBUF = 8
KB = 25
NBLK = 25
EPT = E // NS
K = NBLK * KB

_mesh = plsc.VectorSubcoreMesh(core_axis_name="c", subcore_axis_name="s")


@functools.partial(
    pl.kernel,
    out_type=jax.ShapeDtypeStruct((NC, N, D), jnp.float32),
    mesh=_mesh,
    scratch_types=[
        pltpu.VMEM((2, KB, B), jnp.int32),
        pltpu.VMEM((2, KB, B), jnp.int32),
        pltpu.VMEM((NBUF, B, D), jnp.float32),
        pltpu.VMEM_SHARED((N, D), jnp.float32),
        pltpu.SemaphoreType.DMA((NBUF,)),
        pltpu.SemaphoreType.DMA((2,)),
        pltpu.SemaphoreType.DMA((NBUF,)),
    ],
)
def _agg_kernel(x_hbm, src_hbm, dst_hbm, out_hbm, src_v, dst_v, rows_v, acc_sh,
                gsem, isem, ssem):
    c = lax.axis_index("c")
    s = lax.axis_index("s")
    w = c * NS + s

    pltpu.sync_copy(
        x_hbm.at[pl.ds(s * 624, 624)], acc_sh.at[pl.ds(s * 624, 624)])

    @pl.when(s == 0)
    def _():
        pltpu.sync_copy(
            x_hbm.at[pl.ds(16 * 624, N - 16 * 624)],
            acc_sh.at[pl.ds(16 * 624, N - 16 * 624)],
        )

    plsc.subcore_barrier()

    pltpu.sync_copy(src_hbm.at[w, 0], src_v.at[0])
    pltpu.sync_copy(dst_hbm.at[w, 0], dst_v.at[0])
    for t in range(NBUF - 1):
        pltpu.async_copy(x_hbm.at[src_v.at[0, t]], rows_v.at[t], gsem.at[t])

    def body(j, carry):
        p = j % NBUF
        b = j // KB
        jj = j % KB
        bp = b % 2

        g = j + NBUF - 1
        gp = g % NBUF
        gb = g // KB
        gjj = g % KB
        gbp = gb % 2

        @pl.when(jnp.logical_and(jj == 0, b + 1 < NBLK))
        def _():
            pltpu.async_copy(src_hbm.at[w, b + 1], src_v.at[1 - bp],
                             isem.at[1 - bp])
            pltpu.async_copy(dst_hbm.at[w, b + 1], dst_v.at[1 - bp],
                             isem.at[1 - bp])

        @pl.when(g < K)
        def _():
            @pl.when(gjj == 0)
            def _():
                pltpu.make_async_copy(src_hbm.at[w, gb], src_v.at[gbp],
                                      isem.at[gbp]).wait()
                pltpu.make_async_copy(dst_hbm.at[w, gb], dst_v.at[gbp],
                                      isem.at[gbp]).wait()

            @pl.when(g >= NBUF)
            def _():
                pltpu.make_async_copy(
                    rows_v.at[gp], acc_sh.at[dst_v.at[0, 0]],
                    ssem.at[gp]).wait()

            pltpu.async_copy(x_hbm.at[src_v.at[gbp, gjj]], rows_v.at[gp],
                             gsem.at[gp])

        pltpu.make_async_copy(x_hbm.at[src_v.at[bp, jj]], rows_v.at[p],
                              gsem.at[p]).wait()
        pltpu.async_copy(rows_v.at[p], acc_sh.at[dst_v.at[bp, jj]],
                         ssem.at[p], add=True)
        return carry

    lax.fori_loop(0, K, body, 0, unroll=False)

    for t in range(NBUF):
        pltpu.make_async_copy(rows_v.at[t], acc_sh.at[dst_v.at[0, 0]],
                              ssem.at[t]).wait()

    plsc.subcore_barrier()
    pltpu.sync_copy(
        acc_sh.at[pl.ds(s * 624, 624)],
        out_hbm.at[c, pl.ds(s * 624, 624)],
    )

    @pl.when(s == 0)
    def _():
        pltpu.sync_copy(
            acc_sh.at[pl.ds(16 * 624, N - 16 * 624)],
            out_hbm.at[c, pl.ds(16 * 624, N - 16 * 624)],
        )


BLK = 1000


def _mlp_body(h_ref, w1_ref, b1_ref, w2_ref, b2_ref, o_ref):
    w1 = w1_ref[...]
    b1 = b1_ref[...]
    rf = jnp.maximum(
        jnp.dot(h_ref[0], w1, preferred_element_type=jnp.float32) + b1, 0.0)
    rb = jnp.maximum(
        jnp.dot(h_ref[1], w1, preferred_element_type=jnp.float32) + b1, 0.0)
    o = jnp.dot(rf + rb, w2_ref[...], preferred_element_type=jnp.float32) * 0.5
    o_ref[...] = jnp.maximum(o + b2_ref[...], 0.0)


def kernel(x, edge_index, reverse_edge_index, W1, b1, W2, b2):
    ei = edge_index.astype(jnp.int32)
    rei = reverse_edge_index.astype(jnp.int32)
    src = jnp.concatenate([ei[0], rei[0]]).reshape(NC * NS, NBLK, KB, B)
    dst = jnp.concatenate([ei[1], rei[1]]).reshape(NC * NS, NBLK, KB, B)
    h = _agg_kernel(x, src, dst)

    out = pl.pallas_call(
        _mlp_body,
        grid=(N // BLK,),
        in_specs=[
            pl.BlockSpec((NC, BLK, D), lambda i: (0, i, 0)),
            pl.BlockSpec((D, D), lambda i: (0, 0)),
            pl.BlockSpec((1, D), lambda i: (0, 0)),
            pl.BlockSpec((D, D), lambda i: (0, 0)),
            pl.BlockSpec((1, D), lambda i: (0, 0)),
        ],
        out_specs=pl.BlockSpec((BLK, D), lambda i: (i, 0)),
        out_shape=jax.ShapeDtypeStruct((N, D), jnp.float32),
    )(h, W1, b1.reshape(1, D), W2, b2.reshape(1, D))
    return out
